# baseline (speedup 1.0000x reference)
import functools

import jax
import jax.numpy as jnp
from jax.experimental import pallas as pl
from jax.experimental.pallas import tpu as pltpu

_N_SYS = 2048
_EPS = 1e-6


def _seg_kernel(off_ref, ord_ref, m2cx_ref, m2cy_ref, cc_ref, th_ref,
                b0_ref, b1_ref, cx_ref, cy_ref, g2_ref, gx_ref, gy_ref,
                out_ref, *, rr, cc, bsys):
    s = pl.program_id(0)
    g2 = g2_ref[...]
    gxp = gx_ref[...]
    gyp = gy_ref[...]

    def sis_body(i, carry):
        a, bx, by = carry
        k = ord_ref[i]
        p = m2cx_ref[k] * gxp
        q = m2cy_ref[k] * gyp + p
        u = (g2 + cc_ref[k]) + q
        coef = th_ref[k] * jax.lax.rsqrt(u)
        return a + coef, bx + cx_ref[k] * coef, by + cy_ref[k] * coef

    def pemd_body(i, carry):
        a, bx, by = carry
        k = ord_ref[i]
        p = m2cx_ref[k] * gxp
        q = m2cy_ref[k] * gyp + p
        u = (g2 + cc_ref[k]) + q
        coef = jnp.exp2(b0_ref[k] + b1_ref[k] * jnp.log2(u))
        coef = coef * jax.lax.rsqrt(u)
        return a + coef, bx + cx_ref[k] * coef, by + cy_ref[k] * coef

    zero = jnp.zeros((rr, cc), jnp.float32)
    for j in range(bsys):
        sysid = s * bsys + j
        lo = off_ref[2 * sysid]
        mid = off_ref[2 * sysid + 1]
        hi = off_ref[2 * sysid + 2]
        carry = jax.lax.fori_loop(lo, mid, sis_body, (zero, zero, zero))
        a, bx, by = jax.lax.fori_loop(mid, hi, pemd_body, carry)
        na = 1.0 - a
        out_ref[j, 0] = gxp * na + bx
        out_ref[j, 1] = gyp * na + by


@jax.jit
def kernel(lens_grid, sis_params, pemd_params, sis_idx, pemd_idx):
    hh, ww, _ = lens_grid.shape
    rr = hh * ww // 128
    gx = lens_grid[:, :, 0].reshape(rr, 128)
    gy = lens_grid[:, :, 1].reshape(rr, 128)
    g2 = gx * gx + gy * gy + _EPS

    th = jnp.concatenate([sis_params[:, 0], pemd_params[:, 0]])
    gam = jnp.concatenate([jnp.full(sis_params.shape[:1], 2.0),
                           pemd_params[:, 1]])
    cx = jnp.concatenate([sis_params[:, 1], pemd_params[:, 2]])
    cy = jnp.concatenate([sis_params[:, 2], pemd_params[:, 3]])
    typ = jnp.concatenate([jnp.zeros(sis_idx.shape, jnp.int32),
                           jnp.ones(pemd_idx.shape, jnp.int32)])
    idx2 = 2 * jnp.concatenate([sis_idx, pemd_idx]) + typ

    order = jnp.argsort(idx2).astype(jnp.int32)

    b0 = (gam - 1.0) * jnp.log2(th)
    b1 = 0.5 * (2.0 - gam)
    m2cx = -2.0 * cx
    m2cy = -2.0 * cy
    ccs = cx * cx + cy * cy
    counts = jnp.bincount(idx2, length=2 * _N_SYS)
    off = jnp.concatenate(
        [jnp.zeros((1,), jnp.int32),
         jnp.cumsum(counts).astype(jnp.int32)])

    bsys = 64
    out = pl.pallas_call(
        functools.partial(_seg_kernel, rr=rr, cc=128, bsys=bsys),
        grid=(_N_SYS // bsys,),
        in_specs=[pl.BlockSpec(memory_space=pltpu.SMEM)] * 10 + [
            pl.BlockSpec((rr, 128), lambda s: (0, 0)),
            pl.BlockSpec((rr, 128), lambda s: (0, 0)),
            pl.BlockSpec((rr, 128), lambda s: (0, 0)),
        ],
        out_specs=pl.BlockSpec((bsys, 2, rr, 128), lambda s: (s, 0, 0, 0)),
        out_shape=jax.ShapeDtypeStruct((_N_SYS, 2, rr, 128), jnp.float32),
    )(off, order, m2cx, m2cy, ccs, th, b0, b1, cx, cy, g2, gx, gy)
    return out.reshape(_N_SYS, 2, hh, ww).transpose(0, 2, 3, 1)
